# baseline (speedup 1.0000x reference)
import functools
import math

import jax
import jax.numpy as jnp
from jax import lax
from jax.experimental import pallas as pl
from jax.experimental.pallas import tpu as pltpu
from jax.experimental.pallas import tpu_sc as plsc

B, N, R = 1024, 8192, 5
NW = 32
ROWS_PER_W = B // NW
LANES = 16
CHUNKS = N // LANES
UNROLL = 8
NEG_INF = float("-inf")


def _row_stats(pbuf, lbuf):
    t_init = jnp.full((LANES,), NEG_INF, jnp.float32)
    zeros = jnp.zeros((LANES,), jnp.float32)
    NSETS = 4

    def insert(tset, pm):
        t1, t2, t3 = tset
        hi = jnp.maximum(t1, pm)
        lo = jnp.minimum(t1, pm)
        mid = jnp.maximum(t2, lo)
        lo2 = jnp.minimum(t2, lo)
        return (hi, mid, jnp.maximum(t3, lo2))

    init = (t_init,) * (3 * NSETS) + (zeros, zeros)

    @plsc.parallel_loop(0, CHUNKS // UNROLL, unroll=4, carry=init)
    def carry(i, carry_in):
        sets = [carry_in[3 * s:3 * s + 3] for s in range(NSETS)]
        np0, np1 = carry_in[3 * NSETS], carry_in[3 * NSETS + 1]
        base = i * (LANES * UNROLL)
        for u in range(UNROLL):
            p = pbuf[pl.ds(base + u * LANES, LANES)]
            l = lbuf[pl.ds(base + u * LANES, LANES)]
            keep = l > 0.0
            pm = jnp.where(keep, 0.0, p)
            if u % 2 == 0:
                np0 = np0 + l
            else:
                np1 = np1 + l
            sets[u % NSETS] = insert(sets[u % NSETS], pm)
        return tuple(x for s in sets for x in s) + (np0, np1)
    sets = [carry[3 * s:3 * s + 3] for s in range(NSETS)]
    npos = carry[3 * NSETS] + carry[3 * NSETS + 1]

    t1, t2, t3 = sets[0]
    for s in range(1, NSETS):
        for x in sets[s]:
            t1, t2, t3 = insert((t1, t2, t3), x)

    n_pos = jnp.sum(npos)
    s_pos = n_pos

    vs = []
    for rnd in range(3):
        m = jnp.max(t1)
        vs.append(m)
        if rnd < 2:
            eq = t1 == m
            cs = jnp.cumsum(eq.astype(jnp.int32))
            first = jnp.logical_and(eq, cs == 1)
            t1 = jnp.where(first, t2, t1)
            t2 = jnp.where(first, t3, t2)
    v1, v2, v3 = vs

    nz = (jnp.where(v1 != 0.0, 1.0, 0.0) + jnp.where(v2 != 0.0, 1.0, 0.0)
          + jnp.where(v3 != 0.0, 1.0, 0.0))
    extra_cnt = nz + jnp.maximum(0.0, (3.0 - nz) - n_pos)
    row_sq = s_pos + v1 * v1 + v2 * v2 + v3 * v3
    row_cnt = n_pos + extra_cnt
    return row_sq, row_cnt


def _make_sc_kernel():
    mesh = plsc.VectorSubcoreMesh(
        core_axis_name="c", subcore_axis_name="s", num_cores=2,
        num_subcores=16)

    @functools.partial(
        pl.kernel,
        mesh=mesh,
        compiler_params=pltpu.CompilerParams(needs_layout_passes=False),
        out_type=jax.ShapeDtypeStruct((NW, LANES), jnp.float32),
        scratch_types=[
            pltpu.VMEM((N,), jnp.float32),
            pltpu.VMEM((N,), jnp.float32),
            pltpu.VMEM((N,), jnp.float32),
            pltpu.VMEM((N,), jnp.float32),
            pltpu.VMEM((B * R,), jnp.float32),
            pltpu.VMEM((B * R,), jnp.float32),
            pltpu.VMEM((LANES,), jnp.float32),
            pltpu.SemaphoreType.DMA,
            pltpu.SemaphoreType.DMA,
            pltpu.SemaphoreType.DMA,
            pltpu.SemaphoreType.DMA,
            pltpu.SemaphoreType.DMA,
        ],
    )
    def sck(cp_hbm, cl_hbm, op_hbm, ol_hbm, out_hbm,
            pb0, pb1, lb0, lb1, opb, olb, stage,
            ps0, ps1, ls0, ls1, osem):
        wid = lax.axis_index("s") * 2 + lax.axis_index("c")
        row0 = wid * ROWS_PER_W
        pbufs, lbufs = (pb0, pb1), (lb0, lb1)
        psems, lsems = (ps0, ps1), (ls0, ls1)

        def start(r, par):
            pltpu.async_copy(cp_hbm.at[row0 + r], pbufs[par], psems[par])
            pltpu.async_copy(cl_hbm.at[row0 + r], lbufs[par], lsems[par])

        def wait(par):
            pltpu.make_async_copy(cp_hbm.at[row0], pbufs[par],
                                  psems[par]).wait()
            pltpu.make_async_copy(cl_hbm.at[row0], lbufs[par],
                                  lsems[par]).wait()

        start(0, 0)
        start(1, 1)
        wsq = jnp.float32(0.0)
        wcnt = jnp.float32(0.0)
        for r in range(ROWS_PER_W):
            par = r % 2
            wait(par)
            row_sq, row_cnt = _row_stats(pbufs[par], lbufs[par])
            wsq = wsq + row_sq
            wcnt = wcnt + row_cnt
            if r + 2 < ROWS_PER_W:
                start(r + 2, par)

        @pl.when(wid == 0)
        def _():
            pltpu.async_copy(op_hbm, opb, osem)
            pltpu.async_copy(ol_hbm, olb, osem)
            pltpu.make_async_copy(op_hbm, opb, osem).wait()
            pltpu.make_async_copy(ol_hbm, olb, osem).wait()

            def rbody(i, acc):
                d = opb[pl.ds(i * LANES, LANES)] - olb[pl.ds(i * LANES, LANES)]
                return acc + d * d

            racc = lax.fori_loop(0, (B * R) // LANES, rbody,
                                 jnp.zeros((LANES,), jnp.float32))
            rsum = jnp.sum(racc)
            lane = lax.iota(jnp.int32, LANES)
            stage[...] = jnp.where(
                lane == 0, wsq,
                jnp.where(lane == 1, wcnt, jnp.where(lane == 2, rsum, 0.0)))

        @pl.when(wid != 0)
        def _():
            lane = lax.iota(jnp.int32, LANES)
            stage[...] = jnp.where(
                lane == 0, wsq, jnp.where(lane == 1, wcnt, 0.0))

        pltpu.sync_copy(stage, out_hbm.at[wid])

    return sck


_sc_kernel = _make_sc_kernel()


def kernel(class_pred, offset_pred, class_label, offset_label):
    out = _sc_kernel(class_pred, class_label,
                     offset_pred.reshape(-1), offset_label.reshape(-1))
    sq = jnp.sum(out[:, 0])
    cnt = jnp.sum(out[:, 1])
    rsum = jnp.sum(out[:, 2])
    class_loss = sq / jnp.maximum(cnt, 1.0)
    reg_loss = rsum / jnp.float32(B * R)
    loss = class_loss + reg_loss
    return (loss, class_loss, reg_loss)


# speedup vs baseline: 1.0872x; 1.0872x over previous
import functools
import math

import jax
import jax.numpy as jnp
from jax import lax
from jax.experimental import pallas as pl
from jax.experimental.pallas import tpu as pltpu
from jax.experimental.pallas import tpu_sc as plsc

B, N, R = 1024, 8192, 5
NW = 32
ROWS_PER_W = B // NW
LANES = 16
CHUNKS = N // LANES
UNROLL = 8
RPT = 2
NEG_INF = float("-inf")


def _row_stats(pbuf, lbuf, k):
    t_init = jnp.full((LANES,), NEG_INF, jnp.float32)
    zeros = jnp.zeros((LANES,), jnp.float32)
    NSETS = 4

    def insert(tset, pm):
        t1, t2, t3 = tset
        hi = jnp.maximum(t1, pm)
        lo = jnp.minimum(t1, pm)
        mid = jnp.maximum(t2, lo)
        lo2 = jnp.minimum(t2, lo)
        return (hi, mid, jnp.maximum(t3, lo2))

    def body(i, carry):
        sets = [carry[3 * s:3 * s + 3] for s in range(NSETS)]
        np0, np1 = carry[3 * NSETS], carry[3 * NSETS + 1]
        base = i * (LANES * UNROLL)
        for u in range(UNROLL):
            p = pbuf[k, pl.ds(base + u * LANES, LANES)]
            l = lbuf[k, pl.ds(base + u * LANES, LANES)]
            keep = l > 0.0
            pm = jnp.where(keep, 0.0, p)
            if u % 2 == 0:
                np0 = np0 + l
            else:
                np1 = np1 + l
            sets[u % NSETS] = insert(sets[u % NSETS], pm)
        return tuple(x for s in sets for x in s) + (np0, np1)

    carry = lax.fori_loop(
        0, CHUNKS // UNROLL, body,
        (t_init,) * (3 * NSETS) + (zeros, zeros))
    sets = [carry[3 * s:3 * s + 3] for s in range(NSETS)]
    npos = carry[3 * NSETS] + carry[3 * NSETS + 1]

    t1, t2, t3 = sets[0]
    for s in range(1, NSETS):
        for x in sets[s]:
            t1, t2, t3 = insert((t1, t2, t3), x)

    n_pos = jnp.sum(npos)
    s_pos = n_pos

    vs = []
    for rnd in range(3):
        m = jnp.max(t1)
        vs.append(m)
        if rnd < 2:
            eq = t1 == m
            cs = jnp.cumsum(eq.astype(jnp.int32))
            first = jnp.logical_and(eq, cs == 1)
            t1 = jnp.where(first, t2, t1)
            t2 = jnp.where(first, t3, t2)
    v1, v2, v3 = vs

    nz = (jnp.where(v1 != 0.0, 1.0, 0.0) + jnp.where(v2 != 0.0, 1.0, 0.0)
          + jnp.where(v3 != 0.0, 1.0, 0.0))
    extra_cnt = nz + jnp.maximum(0.0, (3.0 - nz) - n_pos)
    row_sq = s_pos + v1 * v1 + v2 * v2 + v3 * v3
    row_cnt = n_pos + extra_cnt
    return row_sq, row_cnt


def _make_sc_kernel():
    mesh = plsc.VectorSubcoreMesh(
        core_axis_name="c", subcore_axis_name="s", num_cores=2,
        num_subcores=16)

    @functools.partial(
        pl.kernel,
        mesh=mesh,
        compiler_params=pltpu.CompilerParams(needs_layout_passes=False),
        out_type=jax.ShapeDtypeStruct((NW, LANES), jnp.float32),
        scratch_types=[
            pltpu.VMEM((RPT, N), jnp.float32),
            pltpu.VMEM((RPT, N), jnp.float32),
            pltpu.VMEM((RPT, N), jnp.float32),
            pltpu.VMEM((RPT, N), jnp.float32),
            pltpu.VMEM((B * R,), jnp.float32),
            pltpu.VMEM((B * R,), jnp.float32),
            pltpu.VMEM((LANES,), jnp.float32),
            pltpu.SemaphoreType.DMA,
            pltpu.SemaphoreType.DMA,
            pltpu.SemaphoreType.DMA,
            pltpu.SemaphoreType.DMA,
            pltpu.SemaphoreType.DMA,
        ],
    )
    def sck(cp_hbm, cl_hbm, op_hbm, ol_hbm, out_hbm,
            pb0, pb1, lb0, lb1, opb, olb, stage,
            ps0, ps1, ls0, ls1, osem):
        wid = lax.axis_index("s") * 2 + lax.axis_index("c")
        row0 = wid * ROWS_PER_W
        pbufs, lbufs = (pb0, pb1), (lb0, lb1)
        psems, lsems = (ps0, ps1), (ls0, ls1)
        NPAIR = ROWS_PER_W // RPT

        def start(c, par):
            src = pl.ds(row0 + c * RPT, RPT)
            pltpu.async_copy(cp_hbm.at[src], pbufs[par], psems[par])
            pltpu.async_copy(cl_hbm.at[src], lbufs[par], lsems[par])

        def wait(par):
            src = pl.ds(row0, RPT)
            pltpu.make_async_copy(cp_hbm.at[src], pbufs[par],
                                  psems[par]).wait()
            pltpu.make_async_copy(cl_hbm.at[src], lbufs[par],
                                  lsems[par]).wait()

        start(0, 0)
        start(1, 1)
        wsq = jnp.float32(0.0)
        wcnt = jnp.float32(0.0)
        for c in range(NPAIR):
            par = c % 2
            wait(par)
            for k in range(RPT):
                row_sq, row_cnt = _row_stats(pbufs[par], lbufs[par], k)
                wsq = wsq + row_sq
                wcnt = wcnt + row_cnt
            if c + 2 < NPAIR:
                start(c + 2, par)

        @pl.when(wid == 0)
        def _():
            pltpu.async_copy(op_hbm, opb, osem)
            pltpu.async_copy(ol_hbm, olb, osem)
            pltpu.make_async_copy(op_hbm, opb, osem).wait()
            pltpu.make_async_copy(ol_hbm, olb, osem).wait()

            def rbody(i, acc):
                d = opb[pl.ds(i * LANES, LANES)] - olb[pl.ds(i * LANES, LANES)]
                return acc + d * d

            racc = lax.fori_loop(0, (B * R) // LANES, rbody,
                                 jnp.zeros((LANES,), jnp.float32))
            rsum = jnp.sum(racc)
            lane = lax.iota(jnp.int32, LANES)
            stage[...] = jnp.where(
                lane == 0, wsq,
                jnp.where(lane == 1, wcnt, jnp.where(lane == 2, rsum, 0.0)))

        @pl.when(wid != 0)
        def _():
            lane = lax.iota(jnp.int32, LANES)
            stage[...] = jnp.where(
                lane == 0, wsq, jnp.where(lane == 1, wcnt, 0.0))

        pltpu.sync_copy(stage, out_hbm.at[wid])

    return sck


_sc_kernel = _make_sc_kernel()


def kernel(class_pred, offset_pred, class_label, offset_label):
    out = _sc_kernel(class_pred, class_label,
                     offset_pred.reshape(-1), offset_label.reshape(-1))
    sq = jnp.sum(out[:, 0])
    cnt = jnp.sum(out[:, 1])
    rsum = jnp.sum(out[:, 2])
    class_loss = sq / jnp.maximum(cnt, 1.0)
    reg_loss = rsum / jnp.float32(B * R)
    loss = class_loss + reg_loss
    return (loss, class_loss, reg_loss)
